# baseline (speedup 1.0000x reference)
import functools

import jax
import jax.numpy as jnp
import numpy as np
from jax import lax
from jax.experimental import pallas as pl
from jax.experimental.pallas import tpu as pltpu
from jax.experimental.pallas import tpu_sc as plsc

_N = 8192
_B = 16384
_D = 128
_M = _B - _N

_NC = 2
_NS = 16
_NW = _NC * _NS
_CHUNK = 128


def _rotl32(x: np.ndarray, d: int) -> np.ndarray:
    return ((x << np.uint32(d)) | (x >> np.uint32(32 - d))).astype(np.uint32)


def _threefry2x32(k0: int, k1: int, x0: np.ndarray, x1: np.ndarray):
    rot_a, rot_b = (13, 15, 26, 6), (17, 29, 16, 24)
    ks = [np.uint32(k0), np.uint32(k1),
          np.uint32(np.uint32(k0) ^ np.uint32(k1) ^ np.uint32(0x1BD11BDA))]
    x = [(x0 + ks[0]).astype(np.uint32), (x1 + ks[1]).astype(np.uint32)]

    def rounds(x, rots):
        for r in rots:
            x[0] = (x[0] + x[1]).astype(np.uint32)
            x[1] = _rotl32(x[1], r)
            x[1] = (x[0] ^ x[1]).astype(np.uint32)
        return x

    for i, rots in enumerate((rot_a, rot_b, rot_a, rot_b, rot_a)):
        x = rounds(x, rots)
        x[0] = (x[0] + ks[(i + 1) % 3]).astype(np.uint32)
        x[1] = (x[1] + ks[(i + 2) % 3] + np.uint32(i + 1)).astype(np.uint32)
    return x


def _uniform_key1(m: int) -> np.ndarray:
    i = np.arange(m, dtype=np.uint64)
    hi = (i >> np.uint64(32)).astype(np.uint32)
    lo = (i & np.uint64(0xFFFFFFFF)).astype(np.uint32)
    o0, o1 = _threefry2x32(0, 1, hi, lo)
    bits = o0 ^ o1
    return (((bits >> np.uint32(9)) | np.uint32(0x3F800000)).view(np.float32)
            - np.float32(1.0))


def _build_scatter_lists():
    u = _uniform_key1(_M)
    sizes = (_N + np.arange(_M) + 1).astype(np.float32)
    idxs = np.floor(u * sizes).astype(np.int32)
    idxs = np.minimum(idxs, (sizes - 1).astype(np.int32))
    g = np.arange(_N, dtype=np.int32)
    for j in range(_M):
        if idxs[j] < _N:
            g[idxs[j]] = _N + j
    dst = np.nonzero(g != np.arange(_N))[0].astype(np.int32)
    src = g[dst]
    n_chunks = -(-dst.size // _CHUNK)
    pad = n_chunks * _CHUNK - dst.size
    dst = np.concatenate([dst, np.zeros(pad, np.int32)])
    src = np.concatenate([src, np.full(pad, g[0], np.int32)])
    return dst.reshape(n_chunks, _CHUNK), src.reshape(n_chunks, _CHUNK)


_DST, _SRC = _build_scatter_lists()
_NCHUNKS = _DST.shape[0]

_MAX_CHUNKS_PER_W = -(-_NCHUNKS // _NW)

_MESH = plsc.VectorSubcoreMesh(core_axis_name="c", subcore_axis_name="s")


@functools.partial(
    pl.kernel,
    mesh=_MESH,
    out_type=(),
    scratch_types=[
        pltpu.VMEM((1, _CHUNK), jnp.int32),
        pltpu.VMEM((1, _CHUNK), jnp.int32),
        pltpu.VMEM((_CHUNK, _D), jnp.float32),
        pltpu.SemaphoreType.DMA,
        pltpu.SemaphoreType.DMA,
    ],
)
def _sc_fixup(samples_hbm, dst_hbm, src_hbm, buf_ref, dst_v, src_v, rows_v,
              gsem, wsem):
    wid = lax.axis_index("s") * _NC + lax.axis_index("c")
    for k in range(_MAX_CHUNKS_PER_W):
        c = k * _NW
        if c + _NW <= _NCHUNKS:
            handle = True
        else:
            handle = None

        def do_chunk(chunk):
            pltpu.sync_copy(dst_hbm.at[pl.ds(chunk, 1)], dst_v)
            pltpu.sync_copy(src_hbm.at[pl.ds(chunk, 1)], src_v)
            pltpu.async_copy(samples_hbm.at[src_v.at[0]], rows_v, gsem).wait()
            pltpu.async_copy(rows_v, buf_ref.at[dst_v.at[0]], wsem).wait()

        if handle:
            do_chunk(wid + c)
        else:
            @pl.when(wid + c < _NCHUNKS)
            def _():
                do_chunk(wid + c)


def _tc_copy_body(samples_blk, out_blk):
    out_blk[...] = samples_blk[...]


_TC_COPY = pl.pallas_call(
    _tc_copy_body,
    grid=(32,),
    in_specs=[pl.BlockSpec((_N // 32, _D), lambda b: (b, 0))],
    out_specs=pl.BlockSpec((_N // 32, _D), lambda b: (b, 0)),
    out_shape=jax.ShapeDtypeStruct((_N, _D), jnp.float32),
)


def kernel(samples):
    buf = _TC_COPY(samples)
    buf_ref = jax.new_ref(buf)
    _sc_fixup(samples, jnp.asarray(_DST), jnp.asarray(_SRC), buf_ref)
    return jax.freeze(buf_ref)


# speedup vs baseline: 2.2170x; 2.2170x over previous
import functools

import jax
import jax.numpy as jnp
import numpy as np
from jax import lax
from jax.experimental import pallas as pl
from jax.experimental.pallas import tpu as pltpu
from jax.experimental.pallas import tpu_sc as plsc

_N = 8192
_B = 16384
_D = 128
_M = _B - _N

_NC = 2
_NS = 16
_NW = _NC * _NS
_BPW = _N // _NW
_CHUNK = 128
_K = _BPW // _CHUNK


def _rotl32(x: np.ndarray, d: int) -> np.ndarray:
    return ((x << np.uint32(d)) | (x >> np.uint32(32 - d))).astype(np.uint32)


def _threefry2x32(k0: int, k1: int, x0: np.ndarray, x1: np.ndarray):
    rot_a, rot_b = (13, 15, 26, 6), (17, 29, 16, 24)
    ks = [np.uint32(k0), np.uint32(k1),
          np.uint32(np.uint32(k0) ^ np.uint32(k1) ^ np.uint32(0x1BD11BDA))]
    x = [(x0 + ks[0]).astype(np.uint32), (x1 + ks[1]).astype(np.uint32)]

    def rounds(x, rots):
        for r in rots:
            x[0] = (x[0] + x[1]).astype(np.uint32)
            x[1] = _rotl32(x[1], r)
            x[1] = (x[0] ^ x[1]).astype(np.uint32)
        return x

    for i, rots in enumerate((rot_a, rot_b, rot_a, rot_b, rot_a)):
        x = rounds(x, rots)
        x[0] = (x[0] + ks[(i + 1) % 3]).astype(np.uint32)
        x[1] = (x[1] + ks[(i + 2) % 3] + np.uint32(i + 1)).astype(np.uint32)
    return x


def _uniform_key1(m: int) -> np.ndarray:
    i = np.arange(m, dtype=np.uint64)
    hi = (i >> np.uint64(32)).astype(np.uint32)
    lo = (i & np.uint64(0xFFFFFFFF)).astype(np.uint32)
    o0, o1 = _threefry2x32(0, 1, hi, lo)
    bits = o0 ^ o1
    return (((bits >> np.uint32(9)) | np.uint32(0x3F800000)).view(np.float32)
            - np.float32(1.0))


def _build_gather_map() -> np.ndarray:
    u = _uniform_key1(_M)
    sizes = (_N + np.arange(_M) + 1).astype(np.float32)
    idxs = np.floor(u * sizes).astype(np.int32)
    idxs = np.minimum(idxs, (sizes - 1).astype(np.int32))
    g = np.arange(_N, dtype=np.int32)
    for j in range(_M):
        if idxs[j] < _N:
            g[idxs[j]] = _N + j
    return g


_GATHER_MAP = _build_gather_map()

_MESH = plsc.VectorSubcoreMesh(core_axis_name="c", subcore_axis_name="s")


@functools.partial(
    pl.kernel,
    mesh=_MESH,
    out_type=jax.ShapeDtypeStruct((_N, _D), jnp.float32),
    scratch_types=[
        pltpu.VMEM((_BPW,), jnp.int32),
        pltpu.VMEM((_BPW, _D), jnp.float32),
        pltpu.SemaphoreType.DMA,
        pltpu.SemaphoreType.DMA,
    ],
)
def _gather_rows(samples_hbm, idx_hbm, out_hbm, idx_v, rows_v, gsem, wsem):
    wid = lax.axis_index("s") * _NC + lax.axis_index("c")
    base = wid * _BPW
    pltpu.sync_copy(idx_hbm.at[pl.ds(base, _BPW)], idx_v)
    gathers = [
        pltpu.async_copy(
            samples_hbm.at[idx_v.at[pl.ds(j * _CHUNK, _CHUNK)]],
            rows_v.at[pl.ds(j * _CHUNK, _CHUNK)],
            gsem,
        )
        for j in range(_K)
    ]
    writes = []
    for j in range(_K):
        gathers[j].wait()
        writes.append(
            pltpu.async_copy(
                rows_v.at[pl.ds(j * _CHUNK, _CHUNK)],
                out_hbm.at[pl.ds(base + j * _CHUNK, _CHUNK)],
                wsem,
            )
        )
    for w in writes:
        w.wait()


def kernel(samples):
    return _gather_rows(samples, jnp.asarray(_GATHER_MAP))


# speedup vs baseline: 2.2176x; 1.0003x over previous
import functools

import jax
import jax.numpy as jnp
import numpy as np
from jax import lax
from jax.experimental import pallas as pl
from jax.experimental.pallas import tpu as pltpu
from jax.experimental.pallas import tpu_sc as plsc

_N = 8192
_B = 16384
_D = 128
_M = _B - _N

_NC = 2
_NS = 16
_NW = _NC * _NS
_BPW = _N // _NW
_CHUNK = 128
_K = _BPW // _CHUNK


def _rotl32(x: np.ndarray, d: int) -> np.ndarray:
    return ((x << np.uint32(d)) | (x >> np.uint32(32 - d))).astype(np.uint32)


def _threefry2x32(k0: int, k1: int, x0: np.ndarray, x1: np.ndarray):
    rot_a, rot_b = (13, 15, 26, 6), (17, 29, 16, 24)
    ks = [np.uint32(k0), np.uint32(k1),
          np.uint32(np.uint32(k0) ^ np.uint32(k1) ^ np.uint32(0x1BD11BDA))]
    x = [(x0 + ks[0]).astype(np.uint32), (x1 + ks[1]).astype(np.uint32)]

    def rounds(x, rots):
        for r in rots:
            x[0] = (x[0] + x[1]).astype(np.uint32)
            x[1] = _rotl32(x[1], r)
            x[1] = (x[0] ^ x[1]).astype(np.uint32)
        return x

    for i, rots in enumerate((rot_a, rot_b, rot_a, rot_b, rot_a)):
        x = rounds(x, rots)
        x[0] = (x[0] + ks[(i + 1) % 3]).astype(np.uint32)
        x[1] = (x[1] + ks[(i + 2) % 3] + np.uint32(i + 1)).astype(np.uint32)
    return x


def _uniform_key1(m: int) -> np.ndarray:
    i = np.arange(m, dtype=np.uint64)
    hi = (i >> np.uint64(32)).astype(np.uint32)
    lo = (i & np.uint64(0xFFFFFFFF)).astype(np.uint32)
    o0, o1 = _threefry2x32(0, 1, hi, lo)
    bits = o0 ^ o1
    return (((bits >> np.uint32(9)) | np.uint32(0x3F800000)).view(np.float32)
            - np.float32(1.0))


def _build_gather_map() -> np.ndarray:
    u = _uniform_key1(_M)
    sizes = (_N + np.arange(_M) + 1).astype(np.float32)
    idxs = np.floor(u * sizes).astype(np.int32)
    idxs = np.minimum(idxs, (sizes - 1).astype(np.int32))
    g = np.arange(_N, dtype=np.int32)
    for j in range(_M):
        if idxs[j] < _N:
            g[idxs[j]] = _N + j
    g32 = g.reshape(-1, 2, 16).astype(np.uint32)
    packed = g32[:, 0, :] | (g32[:, 1, :] << np.uint32(16))
    return packed.reshape(-1).view(np.int32)


_GATHER_MAP = _build_gather_map()

_MESH = plsc.VectorSubcoreMesh(core_axis_name="c", subcore_axis_name="s")


@functools.partial(
    pl.kernel,
    mesh=_MESH,
    out_type=jax.ShapeDtypeStruct((_N, _D), jnp.float32),
    scratch_types=[
        pltpu.VMEM((_BPW // 2,), jnp.int32),
        pltpu.VMEM((_BPW,), jnp.int32),
        pltpu.VMEM((_BPW, _D), jnp.float32),
        pltpu.SemaphoreType.DMA,
        pltpu.SemaphoreType.DMA,
    ],
)
def _gather_rows(samples_hbm, idx_hbm, out_hbm, idxp_v, idx_v, rows_v,
                 gsem, wsem):
    wid = lax.axis_index("s") * _NC + lax.axis_index("c")
    base = wid * _BPW
    pltpu.sync_copy(idx_hbm.at[pl.ds(wid * (_BPW // 2), _BPW // 2)], idxp_v)
    for k in range(_BPW // 32):
        v = idxp_v[pl.ds(k * 16, 16)]
        idx_v[pl.ds(k * 32, 16)] = v & 0xFFFF
        idx_v[pl.ds(k * 32 + 16, 16)] = lax.shift_right_logical(v, 16)
    gathers = [
        pltpu.async_copy(
            samples_hbm.at[idx_v.at[pl.ds(j * _CHUNK, _CHUNK)]],
            rows_v.at[pl.ds(j * _CHUNK, _CHUNK)],
            gsem,
        )
        for j in range(_K)
    ]
    writes = []
    for j in range(_K):
        gathers[j].wait()
        writes.append(
            pltpu.async_copy(
                rows_v.at[pl.ds(j * _CHUNK, _CHUNK)],
                out_hbm.at[pl.ds(base + j * _CHUNK, _CHUNK)],
                wsem,
            )
        )
    for w in writes:
        w.wait()


def kernel(samples):
    return _gather_rows(samples, jnp.asarray(_GATHER_MAP))
